# Initial kernel scaffold; baseline (speedup 1.0000x reference)
import jax
import jax.numpy as jnp
from jax.experimental import pallas as pl


def kernel(t, edge_index, h0, msg_w1, msg_b1, msg_w2, msg_b2, upd_w1, upd_b1, upd_w2, upd_b2, r_w, r_b):
    raise NotImplementedError("write your pallas kernel here")


# speedup vs baseline: 3.6483x; 3.6483x over previous
import functools

import jax
import jax.numpy as jnp
from jax import lax
from jax.experimental import pallas as pl
from jax.experimental.pallas import tpu as pltpu
from jax.experimental.pallas import tpu_sc as plsc

_N = 50000
_E = 800000
_DH = 16
_DM = 32
_NL = 2
_T = 8

_NC = 2
_NS = 16
_NW = _NC * _NS
_CH = 128
_EPW = 25600
_EPAD = _NW * _EPW
_NCHUNK = _EPW // _CH
_DUMMY = _N
_NTAB = _N
_NACC = 51200
_RSUB = _NACC // _NS
_ZC = _RSUB // _CH

_BN = 2000
_GRID = _N // _BN


def _mesh():
    return plsc.VectorSubcoreMesh(
        core_axis_name="c", subcore_axis_name="s",
        num_cores=_NC, num_subcores=_NS)


def _edge_body(a_tab, b_tab, src, dstg, dsts, out, src_idx, dstg_idx,
               dsts_idx, zrows, srows, acc, sem):
    cid = lax.axis_index("c")
    sid = lax.axis_index("s")
    wid = sid * _NC + cid
    zero = jnp.zeros((16,), jnp.float32)

    def _zfill(e, carry):
        srows[e, pl.ds(0, 16)] = zero
        srows[e, pl.ds(16, 16)] = zero
        return carry

    lax.fori_loop(0, _CH, _zfill, 0)
    my0 = sid * _RSUB

    def _zcopy(m, carry):
        pltpu.sync_copy(srows, acc.at[pl.ds(my0 + m * _CH, _CH)])
        return carry

    lax.fori_loop(0, _ZC, _zcopy, 0)
    plsc.subcore_barrier()

    base = wid * _EPW

    def _chunk(i, carry):
        off = base + i * _CH
        pltpu.sync_copy(src.at[pl.ds(off, _CH)], src_idx)
        pltpu.sync_copy(dstg.at[pl.ds(off, _CH)], dstg_idx)
        pltpu.sync_copy(dsts.at[pl.ds(off, _CH)], dsts_idx)
        pltpu.async_copy(b_tab.at[dstg_idx], zrows, sem).wait()
        pltpu.async_copy(a_tab.at[src_idx], zrows, sem, add=True).wait()

        def _silu(e, c2):
            for hh in (0, 16):
                z = zrows[e, pl.ds(hh, 16)]
                s = z / (1.0 + jnp.exp(-z))
                u = lax.bitcast_convert_type(s, jnp.int32)
                lsb = lax.shift_right_logical(u, jnp.int32(16)) & jnp.int32(1)
                u = (u + jnp.int32(0x7FFF) + lsb) & jnp.int32(-65536)
                srows[e, pl.ds(hh, 16)] = lax.bitcast_convert_type(
                    u, jnp.float32)
            return c2

        lax.fori_loop(0, _CH, _silu, 0)
        pltpu.sync_copy(srows, acc.at[dsts_idx], add=True)
        return carry

    lax.fori_loop(0, _NCHUNK, _chunk, 0)
    plsc.subcore_barrier()
    pltpu.sync_copy(acc.at[pl.ds(my0, _RSUB)], out.at[cid, pl.ds(my0, _RSUB)])


@functools.cache
def _edge_call():
    return pl.kernel(
        _edge_body,
        out_type=jax.ShapeDtypeStruct((_NC, _NACC, _DM), jnp.float32),
        mesh=_mesh(),
        scratch_types=[
            pltpu.VMEM((_CH,), jnp.int32),
            pltpu.VMEM((_CH,), jnp.int32),
            pltpu.VMEM((_CH,), jnp.int32),
            pltpu.VMEM((_CH, _DM), jnp.float32),
            pltpu.VMEM((_CH, _DM), jnp.float32),
            pltpu.VMEM_SHARED((_NACC, _DM), jnp.float32),
            pltpu.SemaphoreType.DMA,
        ],
        compiler_params=pltpu.CompilerParams(use_tc_tiling_on_sc=False),
    )


def _deg_body(dst, out, dst_idx, rows, acc):
    cid = lax.axis_index("c")
    sid = lax.axis_index("s")
    wid = sid * _NC + cid
    zero = jnp.zeros((16,), jnp.float32)

    def _zfill(e, carry):
        rows[e, pl.ds(0, 16)] = zero
        return carry

    lax.fori_loop(0, _CH, _zfill, 0)
    my0 = sid * _RSUB

    def _zcopy(m, carry):
        pltpu.sync_copy(rows, acc.at[pl.ds(my0 + m * _CH, _CH)])
        return carry

    lax.fori_loop(0, _ZC, _zcopy, 0)
    one = jnp.ones((16,), jnp.float32)

    def _ofill(e, carry):
        rows[e, pl.ds(0, 16)] = one
        return carry

    lax.fori_loop(0, _CH, _ofill, 0)
    plsc.subcore_barrier()

    base = wid * _EPW

    def _chunk(i, carry):
        off = base + i * _CH
        pltpu.sync_copy(dst.at[pl.ds(off, _CH)], dst_idx)
        pltpu.sync_copy(rows, acc.at[dst_idx], add=True)
        return carry

    lax.fori_loop(0, _NCHUNK, _chunk, 0)
    plsc.subcore_barrier()
    pltpu.sync_copy(acc.at[pl.ds(my0, _RSUB)], out.at[cid, pl.ds(my0, _RSUB)])


@functools.cache
def _deg_call():
    return pl.kernel(
        _deg_body,
        out_type=jax.ShapeDtypeStruct((_NC, _NACC, 16), jnp.float32),
        mesh=_mesh(),
        scratch_types=[
            pltpu.VMEM((_CH,), jnp.int32),
            pltpu.VMEM((_CH, 16), jnp.float32),
            pltpu.VMEM_SHARED((_NACC, 16), jnp.float32),
        ],
        compiler_params=pltpu.CompilerParams(use_tc_tiling_on_sc=False),
    )


def _silu_tc(x):
    return x * jax.nn.sigmoid(x)


def _dot(a, b):
    return jnp.dot(a, b)


def _pre_body(h_ref, wt, wb, b1, rw, rb, a_out, b_out, y_out):
    h = h_ref[...]
    a_out[...] = _dot(h, wt[...]) + b1[...]
    b_out[...] = _dot(h, wb[...])
    y_out[...] = _dot(h, rw[...]) + rb[...]


def _node_update(g, p0, p1, d0, d1, w2, b2, u1t, u1b, ub1, u2, ub2):
    p = p0[0] + p1[0]
    deg = (d0[0] + d1[0])[:, 0:1]
    agg = jnp.dot(p, w2[...], precision=lax.Precision.HIGHEST) + deg * b2[...]
    pre = _dot(g, u1t[...]) + _dot(agg, u1b[...]) + ub1[...]
    return g + (_dot(_silu_tc(pre), u2[...]) + ub2[...])


def _mid_body(g_ref, p0, p1, d0, d1, w2, b2, u1t, u1b, ub1, u2, ub2,
              nwt, nwb, nb1, g_out, a_out, b_out):
    gnew = _node_update(g_ref[...], p0, p1, d0, d1,
                        w2, b2, u1t, u1b, ub1, u2, ub2)
    g_out[...] = gnew
    a_out[...] = _dot(gnew, nwt[...]) + nb1[...]
    b_out[...] = _dot(gnew, nwb[...])


def _post_body(g_ref, p0, p1, d0, d1, w2, b2, u1t, u1b, ub1, u2, ub2,
               h_ref, dt, rw, rb, *rest, emit_next):
    if emit_next:
        nwt, nwb, nb1, h_out, y_out, a_out, b_out = rest
    else:
        h_out, y_out = rest
    g2 = _node_update(g_ref[...], p0, p1, d0, d1,
                      w2, b2, u1t, u1b, ub1, u2, ub2)
    h = h_ref[...]
    hn = h + dt[0, 0] * (g2 - h)
    h_out[...] = hn
    y_out[...] = _dot(hn, rw[...]) + rb[...]
    if emit_next:
        a_out[...] = _dot(hn, nwt[...]) + nb1[...]
        b_out[...] = _dot(hn, nwb[...])


def _nspec(width):
    return pl.BlockSpec((_BN, width), lambda i: (i, 0))


def _wspec(shape):
    return pl.BlockSpec(shape, lambda i: (0, 0))


def _pspec(part, width):
    return pl.BlockSpec((1, _BN, width), lambda i, _p=part: (_p, i, 0))


_AB_TYPE = jax.ShapeDtypeStruct((_NTAB, _DM), jnp.float32)
_G_TYPE = jax.ShapeDtypeStruct((_N, _DH), jnp.float32)
_Y_TYPE = jax.ShapeDtypeStruct((_N, 1), jnp.float32)


@functools.cache
def _pre_call():
    return pl.pallas_call(
        _pre_body,
        grid=(_GRID,),
        in_specs=[_nspec(_DH), _wspec((_DH, _DM)), _wspec((_DH, _DM)),
                  _wspec((1, _DM)), _wspec((_DH, 1)), _wspec((1, 1))],
        out_specs=[_nspec(_DM), _nspec(_DM), _nspec(1)],
        out_shape=[_AB_TYPE, _AB_TYPE, _Y_TYPE],
    )


def _upd_specs():
    return [_nspec(_DH),
            _pspec(0, _DM), _pspec(1, _DM), _pspec(0, 16), _pspec(1, 16),
            _wspec((_DM, _DM)), _wspec((1, _DM)),
            _wspec((_DH, _DH)), _wspec((_DM, _DH)), _wspec((1, _DH)),
            _wspec((_DH, _DH)), _wspec((1, _DH))]


@functools.cache
def _mid_call():
    return pl.pallas_call(
        _mid_body,
        grid=(_GRID,),
        in_specs=_upd_specs() + [_wspec((_DH, _DM)), _wspec((_DH, _DM)),
                                 _wspec((1, _DM))],
        out_specs=[_nspec(_DH), _nspec(_DM), _nspec(_DM)],
        out_shape=[_G_TYPE, _AB_TYPE, _AB_TYPE],
    )


@functools.cache
def _post_call(emit_next):
    in_specs = _upd_specs() + [_nspec(_DH), _wspec((1, 1)),
                               _wspec((_DH, 1)), _wspec((1, 1))]
    out_specs = [_nspec(_DH), _nspec(1)]
    out_shape = [_G_TYPE, _Y_TYPE]
    if emit_next:
        in_specs += [_wspec((_DH, _DM)), _wspec((_DH, _DM)), _wspec((1, _DM))]
        out_specs += [_nspec(_DM), _nspec(_DM)]
        out_shape += [_AB_TYPE, _AB_TYPE]
    return pl.pallas_call(
        functools.partial(_post_body, emit_next=emit_next),
        grid=(_GRID,),
        in_specs=in_specs,
        out_specs=out_specs,
        out_shape=out_shape,
    )


def kernel(t, edge_index, h0, msg_w1, msg_b1, msg_w2, msg_b2,
           upd_w1, upd_b1, upd_w2, upd_b2, r_w, r_b):
    dts = jnp.diff(t, prepend=t[:1])
    src = jnp.concatenate(
        [edge_index[0].astype(jnp.int32),
         jnp.zeros((_EPAD - _E,), jnp.int32)])
    dst_real = edge_index[1].astype(jnp.int32)
    dstg = jnp.concatenate(
        [dst_real, jnp.zeros((_EPAD - _E,), jnp.int32)])
    dsts = jnp.concatenate(
        [dst_real, jnp.full((_EPAD - _E,), _DUMMY, jnp.int32)])

    def _rtne(x):
        u = lax.bitcast_convert_type(x, jnp.uint32)
        lsb = lax.shift_right_logical(u, jnp.uint32(16)) & jnp.uint32(1)
        u = (u + jnp.uint32(0x7FFF) + lsb) & jnp.uint32(0xFFFF0000)
        return lax.bitcast_convert_type(u, jnp.float32)

    w1t = [msg_w1[l, :_DH] for l in range(_NL)]
    w1b = [msg_w1[l, _DH:] for l in range(_NL)]
    b1 = [msg_b1[l].reshape(1, _DM) for l in range(_NL)]
    w2 = [_rtne(msg_w2[l]) for l in range(_NL)]
    b2 = [msg_b2[l].reshape(1, _DM) for l in range(_NL)]
    u1t = [upd_w1[l, :_DH] for l in range(_NL)]
    u1b = [upd_w1[l, _DH:] for l in range(_NL)]
    ub1 = [upd_b1[l].reshape(1, _DH) for l in range(_NL)]
    u2 = [upd_w2[l] for l in range(_NL)]
    ub2 = [upd_b2[l].reshape(1, _DH) for l in range(_NL)]
    rb = r_b.reshape(1, 1)

    deg = _deg_call()(dsts)

    a_tab, b_tab, y0 = _pre_call()(h0, w1t[0], w1b[0], b1[0], r_w, rb)
    ys = [y0]
    h = h0
    for k in range(1, _T):
        p = _edge_call()(a_tab, b_tab, src, dstg, dsts)
        g1, a_tab, b_tab = _mid_call()(
            h, p, p, deg, deg,
            w2[0], b2[0], u1t[0], u1b[0], ub1[0], u2[0], ub2[0],
            w1t[1], w1b[1], b1[1])
        p = _edge_call()(a_tab, b_tab, src, dstg, dsts)
        dt_k = dts[k].reshape(1, 1)
        if k < _T - 1:
            h, yk, a_tab, b_tab = _post_call(True)(
                g1, p, p, deg, deg,
                w2[1], b2[1], u1t[1], u1b[1], ub1[1], u2[1], ub2[1],
                h, dt_k, r_w, rb, w1t[0], w1b[0], b1[0])
        else:
            h, yk = _post_call(False)(
                g1, p, p, deg, deg,
                w2[1], b2[1], u1t[1], u1b[1], ub1[1], u2[1], ub2[1],
                h, dt_k, r_w, rb)
        ys.append(yk)
    return jnp.concatenate(ys, axis=1).T
